# Initial kernel scaffold; baseline (speedup 1.0000x reference)
import jax
import jax.numpy as jnp
from jax.experimental import pallas as pl


def kernel(node_h, edge_index, edge_h, W_fc, W_edge, b_edge, W_attn):
    raise NotImplementedError("write your pallas kernel here")


# speedup vs baseline: 5.8471x; 5.8471x over previous
import functools

import jax
import jax.numpy as jnp
from jax import lax
from jax.experimental import pallas as pl
from jax.experimental.pallas import tpu as pltpu
from jax.experimental.pallas import tpu_sc as plsc

N = 10000
NP = 10240
E = 320000
EP = 327680
D = 128
ET = EP // 32
K = 128
CHUNKS = ET // K
PAD_DST = N + 16


def _tc_front_body(nh_ref, wt_ref, wsd_ref, ehr_ref, v_ref, z_ref, sd_ref, t2_ref):
    z = jnp.dot(nh_ref[...], wt_ref[...], preferred_element_type=jnp.float32)
    z_ref[...] = z
    sd_ref[...] = jnp.dot(z, wsd_ref[...], preferred_element_type=jnp.float32)
    t2_ref[...] = jnp.dot(ehr_ref[...], v_ref[...], preferred_element_type=jnp.float32)


def _tc_front(node_h, w_fc_t, w_sd, edge_hr, v_blk):
    nb = 1000
    eb = 4000
    return pl.pallas_call(
        _tc_front_body,
        grid=(10,),
        in_specs=[
            pl.BlockSpec((nb, D), lambda i: (i, 0)),
            pl.BlockSpec((D, D), lambda i: (0, 0)),
            pl.BlockSpec((D, 2), lambda i: (0, 0)),
            pl.BlockSpec((eb, D), lambda i: (i, 0)),
            pl.BlockSpec((D, 8), lambda i: (0, 0)),
        ],
        out_specs=[
            pl.BlockSpec((nb, D), lambda i: (i, 0)),
            pl.BlockSpec((nb, 2), lambda i: (i, 0)),
            pl.BlockSpec((eb, 8), lambda i: (i, 0)),
        ],
        out_shape=[
            jax.ShapeDtypeStruct((N, D), jnp.float32),
            jax.ShapeDtypeStruct((N, 2), jnp.float32),
            jax.ShapeDtypeStruct((E // 8, 8), jnp.float32),
        ],
    )(node_h, w_fc_t, w_sd, edge_hr, v_blk)


def _sc_scores_body(s_hbm, d_hbm, t_hbm, src_hbm, dst_hbm, z1_hbm,
                    exp_hbm, den_hbm,
                    s_loc, d_loc, srcb, dstb, tb, exb, den_sh):
    cid = lax.axis_index("c")
    sid = lax.axis_index("s")
    ebase = (sid * 2 + cid) * ET

    pltpu.sync_copy(s_hbm, s_loc)
    pltpu.sync_copy(d_hbm, d_loc)
    pltpu.sync_copy(z1_hbm.at[pl.ds(sid * 640, 640)],
                    den_sh.at[pl.ds(sid * 640, 640)])
    plsc.subcore_barrier()

    def chunk(j, carry):
        base = ebase + j * K
        pltpu.sync_copy(src_hbm.at[pl.ds(base, K)], srcb)
        pltpu.sync_copy(dst_hbm.at[pl.ds(base, K)], dstb)
        pltpu.sync_copy(t_hbm.at[pl.ds(base, K)], tb)
        for g in range(K // 16):
            s16 = plsc.load_gather(s_loc, [srcb[pl.ds(g * 16, 16)]])
            d16 = plsc.load_gather(d_loc, [dstb[pl.ds(g * 16, 16)]])
            a = s16 + d16 + tb[pl.ds(g * 16, 16)]
            e = jnp.maximum(a, 0.01 * a)
            ex = jnp.exp(e)
            gid = base + g * 16 + lax.iota(jnp.int32, 16)
            exb[pl.ds(g * 16, 16)] = jnp.where(gid < E, ex, 0.0)
        pltpu.sync_copy(exb, den_sh.at[dstb], add=True)
        pltpu.sync_copy(exb, exp_hbm.at[pl.ds(base, K)])
        return carry

    lax.fori_loop(0, CHUNKS, chunk, 0)
    plsc.subcore_barrier()
    pltpu.sync_copy(den_sh.at[pl.ds(sid * 640, 640)],
                    den_hbm.at[cid, pl.ds(sid * 640, 640)])


def _sc_scores(s_p, d_p, t_p, src_p, dst_p, zeros1):
    mesh = plsc.VectorSubcoreMesh(core_axis_name="c", subcore_axis_name="s")
    return pl.kernel(
        _sc_scores_body,
        compiler_params=pltpu.CompilerParams(needs_layout_passes=False),
        out_type=[
            jax.ShapeDtypeStruct((EP,), jnp.float32),
            jax.ShapeDtypeStruct((2, NP), jnp.float32),
        ],
        mesh=mesh,
        scratch_types=[
            pltpu.VMEM((NP,), jnp.float32),
            pltpu.VMEM((NP,), jnp.float32),
            pltpu.VMEM((K,), jnp.int32),
            pltpu.VMEM((K,), jnp.int32),
            pltpu.VMEM((K,), jnp.float32),
            pltpu.VMEM((K,), jnp.float32),
            pltpu.VMEM_SHARED((NP,), jnp.float32),
        ],
    )(s_p, d_p, t_p, src_p, dst_p, zeros1)


def _sc_aggr_body(z_hbm, den_hbm, exp_hbm, src_hbm, dst_hbm, z2_hbm,
                  h_hbm,
                  dbuf, dtmp, rows, srcb, dstb, exb, ab, h_sh, sem):
    cid = lax.axis_index("c")
    sid = lax.axis_index("s")
    ebase = (sid * 2 + cid) * ET

    pltpu.sync_copy(den_hbm.at[0], dbuf)
    pltpu.sync_copy(den_hbm.at[1], dtmp)

    def dsum(i, carry):
        dbuf[pl.ds(i * 16, 16)] = dbuf[pl.ds(i * 16, 16)] + dtmp[pl.ds(i * 16, 16)]
        return carry

    lax.fori_loop(0, NP // 16, dsum, 0)

    pltpu.sync_copy(z2_hbm.at[pl.ds(sid * 640, 640), :],
                    h_sh.at[pl.ds(sid * 640, 640), :])
    plsc.subcore_barrier()

    def chunk(j, carry):
        base = ebase + j * K
        pltpu.sync_copy(src_hbm.at[pl.ds(base, K)], srcb)
        pltpu.sync_copy(dst_hbm.at[pl.ds(base, K)], dstb)
        pltpu.sync_copy(exp_hbm.at[pl.ds(base, K)], exb)
        pltpu.async_copy(z_hbm.at[srcb], rows, sem).wait()
        for g in range(K // 16):
            den16 = plsc.load_gather(dbuf, [dstb[pl.ds(g * 16, 16)]])
            ab[pl.ds(g * 16, 16)] = exb[pl.ds(g * 16, 16)] / den16

        def edge_scale(e, c2):
            al = plsc.load_gather(ab, [jnp.zeros((16,), jnp.int32) + e])
            for c in range(D // 16):
                rows[e, pl.ds(c * 16, 16)] = rows[e, pl.ds(c * 16, 16)] * al
            return c2

        lax.fori_loop(0, K, edge_scale, 0)
        pltpu.sync_copy(rows, h_sh.at[dstb], add=True)
        return carry

    lax.fori_loop(0, CHUNKS, chunk, 0)
    plsc.subcore_barrier()
    pltpu.sync_copy(h_sh.at[pl.ds(sid * 640, 640), :],
                    h_hbm.at[cid, pl.ds(sid * 640, 640), :])


def _sc_aggr(z, den_p, exp_p, src_p, dst_p, zeros2):
    mesh = plsc.VectorSubcoreMesh(core_axis_name="c", subcore_axis_name="s")
    return pl.kernel(
        _sc_aggr_body,
        compiler_params=pltpu.CompilerParams(needs_layout_passes=False),
        out_type=jax.ShapeDtypeStruct((2, NP, D), jnp.float32),
        mesh=mesh,
        scratch_types=[
            pltpu.VMEM((NP,), jnp.float32),
            pltpu.VMEM((NP,), jnp.float32),
            pltpu.VMEM((K, D), jnp.float32),
            pltpu.VMEM((K,), jnp.int32),
            pltpu.VMEM((K,), jnp.int32),
            pltpu.VMEM((K,), jnp.float32),
            pltpu.VMEM((K,), jnp.float32),
            pltpu.VMEM_SHARED((NP, D), jnp.float32),
            pltpu.SemaphoreType.DMA,
        ],
    )(z, den_p, exp_p, src_p, dst_p, zeros2)


def _tc_blend_body(nh_ref, hp_ref, o_ref):
    o_ref[...] = 0.5 * nh_ref[...] + 0.5 * (hp_ref[0] + hp_ref[1])


def _tc_blend(node_h, h_part):
    nb = 1000
    return pl.pallas_call(
        _tc_blend_body,
        grid=(10,),
        in_specs=[
            pl.BlockSpec((nb, D), lambda i: (i, 0)),
            pl.BlockSpec((2, nb, D), lambda i: (0, i, 0)),
        ],
        out_specs=pl.BlockSpec((nb, D), lambda i: (i, 0)),
        out_shape=jax.ShapeDtypeStruct((N, D), jnp.float32),
    )(node_h, h_part)


def kernel(node_h, edge_index, edge_h, W_fc, W_edge, b_edge, W_attn):
    f32 = jnp.float32
    src = edge_index[0].astype(jnp.int32)
    dst = edge_index[1].astype(jnp.int32)

    w1 = W_attn[0, :D]
    w2 = W_attn[0, D:2 * D]
    w3 = W_attn[0, 2 * D:]
    v3 = W_edge.T @ w3
    c3 = jnp.dot(b_edge, w3)

    edge_hr = edge_h.reshape(E // 8, D)
    v_blk = jnp.zeros((D, 8), f32).at[jnp.arange(D), jnp.arange(D) // 16].set(
        jnp.tile(v3, 8))
    w_sd = jnp.stack([w1, w2], axis=1)

    z, sd, t2 = _tc_front(node_h, W_fc.T, w_sd, edge_hr, v_blk)

    s_p = jnp.pad(sd[:, 0] + c3, (0, NP - N))
    d_p = jnp.pad(sd[:, 1], (0, NP - N))
    t_p = jnp.pad(t2.reshape(E), (0, EP - E))
    src_p = jnp.pad(src, (0, EP - E))
    dst_p = jnp.pad(dst, (0, EP - E), constant_values=PAD_DST)
    zeros1 = jnp.zeros((NP,), f32)
    zeros2 = jnp.zeros((NP, D), f32)

    exp_p, den_p = _sc_scores(s_p, d_p, t_p, src_p, dst_p, zeros1)
    h_part = _sc_aggr(z, den_p, exp_p, src_p, dst_p, zeros2)
    return _tc_blend(node_h, h_part)
